# Initial kernel scaffold; baseline (speedup 1.0000x reference)
import jax
import jax.numpy as jnp
from jax.experimental import pallas as pl


def kernel(points, tables):
    raise NotImplementedError("write your pallas kernel here")


# speedup vs baseline: 10.9860x; 10.9860x over previous
import dataclasses
import functools

import jax
import jax.numpy as jnp
import numpy as np
from jax import lax
from jax.experimental import pallas as pl
from jax.experimental.pallas import tpu as pltpu
from jax.experimental.pallas import tpu_sc as plsc

_NUM_LEVELS = 16
_LOG2_HASHMAP = 19
_TABLE_SIZE = 1 << _LOG2_HASHMAP
_MASK = _TABLE_SIZE - 1
_BASE_RES = 16
_FINEST_RES = 2048
_P1 = np.uint32(2654435761).astype(np.int32)
_P2 = np.int32(805459861)

_NC = 2
_NS = 16
_NW = _NC * _NS
_LANES = 16
_GW = 128


def _loop32(n, body):
    lax.fori_loop(jnp.int32(0), jnp.int32(n), lambda i, _: body(i), None)


def _resolutions():
    res = []
    for i in range(_NUM_LEVELS):
        r = int(np.floor(_BASE_RES * np.exp(
            i * np.log(_FINEST_RES / _BASE_RES) / (_NUM_LEVELS - 1))))
        res.append(r)
    return res


def _hash_encode_sc(px, py, pz, table_w8, n_points, chunk):
    pw = n_points // _NW
    n_chunks = pw // chunk
    idx_len = chunk * _NUM_LEVELS
    n_gw = idx_len // _GW
    res_m1 = [np.float32(r - 1) for r in _resolutions()]

    mesh = plsc.VectorSubcoreMesh(core_axis_name="core",
                                  subcore_axis_name="subcore",
                                  num_cores=_NC, num_subcores=_NS)
    cp = pltpu.CompilerParams()
    if "needs_layout_passes" in pltpu.CompilerParams.__dataclass_fields__:
        cp = dataclasses.replace(cp, needs_layout_passes=False)
    if "use_tc_tiling_on_sc" in pltpu.CompilerParams.__dataclass_fields__:
        cp = dataclasses.replace(cp, use_tc_tiling_on_sc=False)

    @functools.partial(
        pl.kernel,
        out_type=jax.ShapeDtypeStruct((n_points * _NUM_LEVELS * 2,),
                                      jnp.float32),
        mesh=mesh,
        compiler_params=cp,
        scratch_types=[
            pltpu.VMEM((chunk,), jnp.float32),
            pltpu.VMEM((chunk,), jnp.float32),
            pltpu.VMEM((chunk,), jnp.float32),
            pltpu.VMEM((n_gw, _GW), jnp.int32),
            pltpu.VMEM((idx_len,), jnp.int32),
            pltpu.VMEM((idx_len, 8), jnp.float32),
            pltpu.VMEM((idx_len * 2,), jnp.float32),
            pltpu.SemaphoreType.DMA,
        ],
    )
    def run(px_hbm, py_hbm, pz_hbm, tab_hbm, out_hbm,
            px_v, py_v, pz_v, idx_v, col_v, rows_v, out_v, sem):
        wid = (lax.axis_index("subcore").astype(jnp.int32) * np.int32(_NC)
               + lax.axis_index("core").astype(jnp.int32))
        lane = lax.iota(jnp.int32, 16)
        lane16 = lane * np.int32(16)
        lane2 = lane * np.int32(2)

        @functools.partial(_loop32, n_chunks)
        def _chunk(ci):
            base = wid * np.int32(pw) + ci * np.int32(chunk)
            pltpu.sync_copy(px_hbm.at[pl.ds(base, chunk)], px_v)
            pltpu.sync_copy(py_hbm.at[pl.ds(base, chunk)], py_v)
            pltpu.sync_copy(pz_hbm.at[pl.ds(base, chunk)], pz_v)

            @functools.partial(_loop32, chunk // _LANES)
            def _grp(g):
                goff = g * np.int32(_LANES)
                x = px_v[pl.ds(goff, _LANES)]
                y = py_v[pl.ds(goff, _LANES)]
                z = pz_v[pl.ds(goff, _LANES)]
                one = jnp.float32(1.0)
                half = jnp.float32(0.5)
                zero = jnp.float32(0.0)
                pnx = jnp.minimum(jnp.maximum((x + one) * half, zero), one)
                pny = jnp.minimum(jnp.maximum((y + one) * half, zero), one)
                pnz = jnp.minimum(jnp.maximum((z + one) * half, zero), one)
                gbase = g * np.int32(_LANES * _NUM_LEVELS)
                for lvl in range(_NUM_LEVELS):
                    gx = (pnx * res_m1[lvl]).astype(jnp.int32)
                    gy = (pny * res_m1[lvl]).astype(jnp.int32)
                    gz = (pnz * res_m1[lvl]).astype(jnp.int32)
                    h = gx ^ (gy * _P1) ^ (gz * _P2)
                    h = (h & jnp.int32(_MASK)) + jnp.int32(lvl * _TABLE_SIZE)
                    q = lax.shift_right_logical(h, jnp.int32(2))
                    c2 = lax.shift_left(h & jnp.int32(3), jnp.int32(1))
                    pos = lane16 + (gbase + np.int32(lvl))
                    row = lax.shift_right_logical(pos, jnp.int32(7))
                    col = pos & jnp.int32(_GW - 1)
                    plsc.store_scatter(idx_v, [row, col], q)
                    plsc.store_scatter(col_v, [pos], c2)

            @functools.partial(_loop32, n_gw)
            def _fire(ji):
                pltpu.async_copy(
                    tab_hbm.at[idx_v.at[ji]],
                    rows_v.at[pl.ds(ji * np.int32(_GW), _GW)], sem)

            @functools.partial(_loop32, n_gw)
            def _drain(ji):
                pltpu.make_async_copy(
                    tab_hbm.at[idx_v.at[ji]],
                    rows_v.at[pl.ds(ji * np.int32(_GW), _GW)], sem).wait()

            @functools.partial(_loop32, idx_len // _LANES)
            def _compact(j):
                j16 = j * np.int32(_LANES)
                rowj = lane + j16
                colj = col_v[pl.ds(j16, _LANES)]
                f0 = plsc.load_gather(rows_v, [rowj, colj])
                f1 = plsc.load_gather(rows_v, [rowj, colj + jnp.int32(1)])
                p0 = lane2 + j * np.int32(2 * _LANES)
                plsc.store_scatter(out_v, [p0], f0)
                plsc.store_scatter(out_v, [p0 + jnp.int32(1)], f1)

            pltpu.sync_copy(
                out_v,
                out_hbm.at[pl.ds(base * np.int32(2 * _NUM_LEVELS),
                                 idx_len * 2)])

    return run(px, py, pz, table_w8)


def kernel(points, tables):
    n = points.shape[0]
    px = points[:, 0]
    py = points[:, 1]
    pz = points[:, 2]
    table_w8 = tables.reshape(_NUM_LEVELS * _TABLE_SIZE // 4, 8)
    out = _hash_encode_sc(px, py, pz, table_w8, n, chunk=512)
    return out.reshape(n, _NUM_LEVELS * 2)


# speedup vs baseline: 45.6393x; 4.1543x over previous
import dataclasses
import functools

import jax
import jax.numpy as jnp
import numpy as np
from jax import lax
from jax.experimental import pallas as pl
from jax.experimental.pallas import tpu as pltpu
from jax.experimental.pallas import tpu_sc as plsc

_NUM_LEVELS = 16
_LOG2_HASHMAP = 19
_TABLE_SIZE = 1 << _LOG2_HASHMAP
_MASK = _TABLE_SIZE - 1
_BASE_RES = 16
_FINEST_RES = 2048
_P1 = np.uint32(2654435761).astype(np.int32)
_P2 = np.int32(805459861)

_NC = 2
_NS = 16
_NW = _NC * _NS
_LANES = 16
_GW = 128
_LROWS = _TABLE_SIZE * 2 // 8


def _loop32(n, body):
    lax.fori_loop(jnp.int32(0), jnp.int32(n), lambda i, _: body(i), None)


def _resolutions():
    res = []
    for i in range(_NUM_LEVELS):
        r = int(np.floor(_BASE_RES * np.exp(
            i * np.log(_FINEST_RES / _BASE_RES) / (_NUM_LEVELS - 1))))
        res.append(r)
    return res


def _hash_encode_sc(px, py, pz, table_w8, n_points, chunk):
    pw = n_points // _NW
    n_chunks = pw // chunk
    idx_len = chunk * _NUM_LEVELS
    n_rows = idx_len * 2
    n_gw = n_rows // _GW
    res_m1 = [np.float32(r - 1) for r in _resolutions()]

    mesh = plsc.VectorSubcoreMesh(core_axis_name="core",
                                  subcore_axis_name="subcore",
                                  num_cores=_NC, num_subcores=_NS)
    cp = pltpu.CompilerParams()
    if "needs_layout_passes" in pltpu.CompilerParams.__dataclass_fields__:
        cp = dataclasses.replace(cp, needs_layout_passes=False)
    if "use_tc_tiling_on_sc" in pltpu.CompilerParams.__dataclass_fields__:
        cp = dataclasses.replace(cp, use_tc_tiling_on_sc=False)

    @functools.partial(
        pl.kernel,
        out_type=jax.ShapeDtypeStruct((n_points * _NUM_LEVELS * 2,),
                                      jnp.float32),
        mesh=mesh,
        compiler_params=cp,
        scratch_types=[
            pltpu.VMEM((chunk,), jnp.float32),
            pltpu.VMEM((chunk,), jnp.float32),
            pltpu.VMEM((chunk,), jnp.float32),
            pltpu.VMEM((n_gw, _GW), jnp.int32),
            pltpu.VMEM((idx_len,), jnp.int32),
            pltpu.VMEM((n_rows, 8), jnp.float32),
            pltpu.VMEM((idx_len * 2,), jnp.float32),
            pltpu.SemaphoreType.DMA,
        ],
    )
    def run(px_hbm, py_hbm, pz_hbm, tab_hbm, out_hbm,
            px_v, py_v, pz_v, idx_v, col_v, rows_v, out_v, sem):
        wid = (lax.axis_index("subcore").astype(jnp.int32) * np.int32(_NC)
               + lax.axis_index("core").astype(jnp.int32))
        lane = lax.iota(jnp.int32, 16)
        lane2 = lane * np.int32(2)
        lane16 = lane * np.int32(16)
        lane32 = lane * np.int32(32)

        @functools.partial(_loop32, n_chunks)
        def _chunk(ci):
            base = wid * np.int32(pw) + ci * np.int32(chunk)
            pltpu.sync_copy(px_hbm.at[pl.ds(base, chunk)], px_v)
            pltpu.sync_copy(py_hbm.at[pl.ds(base, chunk)], py_v)
            pltpu.sync_copy(pz_hbm.at[pl.ds(base, chunk)], pz_v)

            @functools.partial(_loop32, chunk // _LANES)
            def _grp(g):
                goff = g * np.int32(_LANES)
                x = px_v[pl.ds(goff, _LANES)]
                y = py_v[pl.ds(goff, _LANES)]
                z = pz_v[pl.ds(goff, _LANES)]
                one = jnp.float32(1.0)
                half = jnp.float32(0.5)
                zero = jnp.float32(0.0)
                pnx = jnp.minimum(jnp.maximum((x + one) * half, zero), one)
                pny = jnp.minimum(jnp.maximum((y + one) * half, zero), one)
                pnz = jnp.minimum(jnp.maximum((z + one) * half, zero), one)
                gb16 = g * np.int32(_LANES * _NUM_LEVELS)
                gb32 = g * np.int32(_LANES * _NUM_LEVELS * 2)
                for lvl in range(_NUM_LEVELS):
                    gx = (pnx * res_m1[lvl]).astype(jnp.int32)
                    gy = (pny * res_m1[lvl]).astype(jnp.int32)
                    gz = (pnz * res_m1[lvl]).astype(jnp.int32)
                    h = gx ^ (gy * _P1) ^ (gz * _P2)
                    h = h & jnp.int32(_MASK)
                    r0 = (lax.shift_right_logical(h, jnp.int32(3))
                          + lax.shift_left(
                              lax.shift_right_logical(h, jnp.int32(7)),
                              jnp.int32(4))
                          + jnp.int32(lvl * _LROWS))
                    r1 = r0 + jnp.int32(16)
                    off = h & jnp.int32(7)
                    pos2 = lane32 + (gb32 + np.int32(2 * lvl))
                    row = lax.shift_right_logical(pos2, jnp.int32(7))
                    col = pos2 & jnp.int32(_GW - 1)
                    plsc.store_scatter(idx_v, [row, col], r0)
                    plsc.store_scatter(idx_v, [row, col + jnp.int32(1)], r1)
                    pos = lane16 + (gb16 + np.int32(lvl))
                    plsc.store_scatter(col_v, [pos], off)

            @functools.partial(_loop32, n_gw)
            def _fire(ji):
                pltpu.async_copy(
                    tab_hbm.at[idx_v.at[ji]],
                    rows_v.at[pl.ds(ji * np.int32(_GW), _GW)], sem)

            @functools.partial(_loop32, n_gw)
            def _drain(ji):
                pltpu.make_async_copy(
                    tab_hbm.at[idx_v.at[ji]],
                    rows_v.at[pl.ds(ji * np.int32(_GW), _GW)], sem).wait()

            @functools.partial(_loop32, idx_len // _LANES)
            def _compact(j):
                j16 = j * np.int32(_LANES)
                rowj = lane2 + j * np.int32(2 * _LANES)
                colj = col_v[pl.ds(j16, _LANES)]
                f0 = plsc.load_gather(rows_v, [rowj, colj])
                f1 = plsc.load_gather(rows_v, [rowj + jnp.int32(1), colj])
                p0 = lane2 + j * np.int32(2 * _LANES)
                plsc.store_scatter(out_v, [p0], f0)
                plsc.store_scatter(out_v, [p0 + jnp.int32(1)], f1)

            pltpu.sync_copy(
                out_v,
                out_hbm.at[pl.ds(base * np.int32(2 * _NUM_LEVELS),
                                 idx_len * 2)])

    return run(px, py, pz, table_w8)


def kernel(points, tables):
    n = points.shape[0]
    px = points[:, 0]
    py = points[:, 1]
    pz = points[:, 2]
    table_w8 = (tables.reshape(_NUM_LEVELS, _TABLE_SIZE // _GW, _GW, 2)
                .transpose(0, 1, 3, 2)
                .reshape(_NUM_LEVELS * _TABLE_SIZE * 2 // 8, 8))
    out = _hash_encode_sc(px, py, pz, table_w8, n, chunk=256)
    return out.reshape(n, _NUM_LEVELS * 2)


# speedup vs baseline: 63.6634x; 1.3949x over previous
import dataclasses
import functools

import jax
import jax.numpy as jnp
import numpy as np
from jax import lax
from jax.experimental import pallas as pl
from jax.experimental.pallas import tpu as pltpu
from jax.experimental.pallas import tpu_sc as plsc

_NUM_LEVELS = 16
_LOG2_HASHMAP = 19
_TABLE_SIZE = 1 << _LOG2_HASHMAP
_MASK = _TABLE_SIZE - 1
_BASE_RES = 16
_FINEST_RES = 2048
_P1 = np.uint32(2654435761).astype(np.int32)
_P2 = np.int32(805459861)

_NC = 2
_NS = 16
_NW = _NC * _NS
_LANES = 16
_GW = 128
_LROWS = _TABLE_SIZE * 2 // 8


def _loop32(n, body):
    lax.fori_loop(jnp.int32(0), jnp.int32(n), lambda i, _: body(i), None)


def _resolutions():
    res = []
    for i in range(_NUM_LEVELS):
        r = int(np.floor(_BASE_RES * np.exp(
            i * np.log(_FINEST_RES / _BASE_RES) / (_NUM_LEVELS - 1))))
        res.append(r)
    return res


def _hash_encode_sc(px, py, pz, table_w8, n_points, chunk):
    pw = n_points // _NW
    n_chunks = pw // chunk
    idx_len = chunk * _NUM_LEVELS
    n_rows = idx_len * 2
    n_gw = n_rows // _GW
    res_m1 = [np.float32(r - 1) for r in _resolutions()]

    mesh = plsc.VectorSubcoreMesh(core_axis_name="core",
                                  subcore_axis_name="subcore",
                                  num_cores=_NC, num_subcores=_NS)
    cp = pltpu.CompilerParams()
    if "needs_layout_passes" in pltpu.CompilerParams.__dataclass_fields__:
        cp = dataclasses.replace(cp, needs_layout_passes=False)
    if "use_tc_tiling_on_sc" in pltpu.CompilerParams.__dataclass_fields__:
        cp = dataclasses.replace(cp, use_tc_tiling_on_sc=False)

    @functools.partial(
        pl.kernel,
        out_type=jax.ShapeDtypeStruct((n_points * _NUM_LEVELS * 2,),
                                      jnp.float32),
        mesh=mesh,
        compiler_params=cp,
        scratch_types=[
            [pltpu.VMEM((chunk,), jnp.float32)] * 2,
            [pltpu.VMEM((chunk,), jnp.float32)] * 2,
            [pltpu.VMEM((chunk,), jnp.float32)] * 2,
            [pltpu.VMEM((n_gw, _GW), jnp.int32)] * 2,
            [pltpu.VMEM((idx_len,), jnp.int32)] * 2,
            [pltpu.VMEM((n_rows, 8), jnp.float32)] * 2,
            [pltpu.VMEM((idx_len * 2,), jnp.float32)] * 2,
            [pltpu.SemaphoreType.DMA] * 2,
            [pltpu.SemaphoreType.DMA] * 2,
        ],
    )
    def run(px_hbm, py_hbm, pz_hbm, tab_hbm, out_hbm,
            px_v, py_v, pz_v, idx_v, col_v, rows_v, out_v, gsem, osem):
        wid = (lax.axis_index("subcore").astype(jnp.int32) * np.int32(_NC)
               + lax.axis_index("core").astype(jnp.int32))
        lane = lax.iota(jnp.int32, 16)
        lane2 = lane * np.int32(2)
        lane16 = lane * np.int32(16)
        lane32 = lane * np.int32(32)

        def stage_a(ci, b):
            base = wid * np.int32(pw) + ci * np.int32(chunk)
            pltpu.sync_copy(px_hbm.at[pl.ds(base, chunk)], px_v[b])
            pltpu.sync_copy(py_hbm.at[pl.ds(base, chunk)], py_v[b])
            pltpu.sync_copy(pz_hbm.at[pl.ds(base, chunk)], pz_v[b])

            @functools.partial(_loop32, chunk // _LANES)
            def _grp(g):
                goff = g * np.int32(_LANES)
                x = px_v[b][pl.ds(goff, _LANES)]
                y = py_v[b][pl.ds(goff, _LANES)]
                z = pz_v[b][pl.ds(goff, _LANES)]
                one = jnp.float32(1.0)
                half = jnp.float32(0.5)
                zero = jnp.float32(0.0)
                pnx = jnp.minimum(jnp.maximum((x + one) * half, zero), one)
                pny = jnp.minimum(jnp.maximum((y + one) * half, zero), one)
                pnz = jnp.minimum(jnp.maximum((z + one) * half, zero), one)
                gb16 = g * np.int32(_LANES * _NUM_LEVELS)
                gb32 = g * np.int32(_LANES * _NUM_LEVELS * 2)
                for lvl in range(_NUM_LEVELS):
                    gx = (pnx * res_m1[lvl]).astype(jnp.int32)
                    gy = (pny * res_m1[lvl]).astype(jnp.int32)
                    gz = (pnz * res_m1[lvl]).astype(jnp.int32)
                    h = gx ^ (gy * _P1) ^ (gz * _P2)
                    h = h & jnp.int32(_MASK)
                    r0 = (lax.shift_right_logical(h, jnp.int32(3))
                          + lax.shift_left(
                              lax.shift_right_logical(h, jnp.int32(7)),
                              jnp.int32(4))
                          + jnp.int32(lvl * _LROWS))
                    r1 = r0 + jnp.int32(16)
                    off = h & jnp.int32(7)
                    pos2 = lane32 + (gb32 + np.int32(2 * lvl))
                    row = lax.shift_right_logical(pos2, jnp.int32(7))
                    col = pos2 & jnp.int32(_GW - 1)
                    plsc.store_scatter(idx_v[b], [row, col], r0)
                    plsc.store_scatter(idx_v[b], [row, col + jnp.int32(1)], r1)
                    pos = lane16 + (gb16 + np.int32(lvl))
                    plsc.store_scatter(col_v[b], [pos], off)

            @functools.partial(_loop32, n_gw)
            def _fire(ji):
                pltpu.async_copy(
                    tab_hbm.at[idx_v[b].at[ji]],
                    rows_v[b].at[pl.ds(ji * np.int32(_GW), _GW)], gsem[b])

        def stage_b(ci, b):
            base = wid * np.int32(pw) + ci * np.int32(chunk)

            @functools.partial(_loop32, n_gw)
            def _drain(ji):
                pltpu.make_async_copy(
                    tab_hbm.at[idx_v[b].at[ji]],
                    rows_v[b].at[pl.ds(ji * np.int32(_GW), _GW)],
                    gsem[b]).wait()

            @pl.when(ci >= jnp.int32(2))
            def _():
                pltpu.make_async_copy(
                    out_v[b],
                    out_hbm.at[pl.ds(jnp.int32(0), idx_len * 2)],
                    osem[b]).wait()

            @functools.partial(_loop32, idx_len // _LANES)
            def _compact(j):
                j16 = j * np.int32(_LANES)
                rowj = lane2 + j * np.int32(2 * _LANES)
                colj = col_v[b][pl.ds(j16, _LANES)]
                f0 = plsc.load_gather(rows_v[b], [rowj, colj])
                f1 = plsc.load_gather(rows_v[b], [rowj + jnp.int32(1), colj])
                plsc.store_scatter(out_v[b], [rowj], f0)
                plsc.store_scatter(out_v[b], [rowj + jnp.int32(1)], f1)

            pltpu.async_copy(
                out_v[b],
                out_hbm.at[pl.ds(base * np.int32(2 * _NUM_LEVELS),
                                 idx_len * 2)], osem[b])

        stage_a(jnp.int32(0), 0)

        @functools.partial(_loop32, n_chunks // 2)
        def _pair(i):
            ci0 = i * np.int32(2)
            ci1 = ci0 + jnp.int32(1)
            stage_a(ci1, 1)
            stage_b(ci0, 0)

            @pl.when(ci1 + jnp.int32(1) < jnp.int32(n_chunks))
            def _():
                stage_a(ci1 + jnp.int32(1), 0)

            stage_b(ci1, 1)

        for b in range(2):
            pltpu.make_async_copy(
                out_v[b], out_hbm.at[pl.ds(jnp.int32(0), idx_len * 2)],
                osem[b]).wait()

    return run(px, py, pz, table_w8)


def kernel(points, tables):
    n = points.shape[0]
    px = points[:, 0]
    py = points[:, 1]
    pz = points[:, 2]
    table_w8 = (tables.reshape(_NUM_LEVELS, _TABLE_SIZE // _GW, _GW, 2)
                .transpose(0, 1, 3, 2)
                .reshape(_NUM_LEVELS * _TABLE_SIZE * 2 // 8, 8))
    out = _hash_encode_sc(px, py, pz, table_w8, n, chunk=128)
    return out.reshape(n, _NUM_LEVELS * 2)


# speedup vs baseline: 70.7422x; 1.1112x over previous
import dataclasses
import functools

import jax
import jax.numpy as jnp
import numpy as np
from jax import lax
from jax.experimental import pallas as pl
from jax.experimental.pallas import tpu as pltpu
from jax.experimental.pallas import tpu_sc as plsc

_NUM_LEVELS = 16
_LOG2_HASHMAP = 19
_TABLE_SIZE = 1 << _LOG2_HASHMAP
_MASK = _TABLE_SIZE - 1
_BASE_RES = 16
_FINEST_RES = 2048
_P1 = np.uint32(2654435761).astype(np.int32)
_P2 = np.int32(805459861)

_NC = 2
_NS = 16
_NW = _NC * _NS
_LANES = 16
_GW = 128
_LROWS = _TABLE_SIZE * 2 // 8


def _loop32(n, body):
    lax.fori_loop(jnp.int32(0), jnp.int32(n), lambda i, _: body(i), None)


def _resolutions():
    res = []
    for i in range(_NUM_LEVELS):
        r = int(np.floor(_BASE_RES * np.exp(
            i * np.log(_FINEST_RES / _BASE_RES) / (_NUM_LEVELS - 1))))
        res.append(r)
    return res


def _hash_encode_sc(px, py, pz, table_w8, n_points, chunk):
    pw = n_points // _NW
    n_chunks = pw // chunk
    idx_len = chunk * _NUM_LEVELS
    n_rows = idx_len * 2
    n_gw = n_rows // _GW
    res_m1 = [np.float32(r - 1) for r in _resolutions()]

    mesh = plsc.VectorSubcoreMesh(core_axis_name="core",
                                  subcore_axis_name="subcore",
                                  num_cores=_NC, num_subcores=_NS)
    cp = pltpu.CompilerParams()
    if "needs_layout_passes" in pltpu.CompilerParams.__dataclass_fields__:
        cp = dataclasses.replace(cp, needs_layout_passes=False)
    if "use_tc_tiling_on_sc" in pltpu.CompilerParams.__dataclass_fields__:
        cp = dataclasses.replace(cp, use_tc_tiling_on_sc=False)

    @functools.partial(
        pl.kernel,
        out_type=jax.ShapeDtypeStruct((n_points * _NUM_LEVELS * 2,),
                                      jnp.float32),
        mesh=mesh,
        compiler_params=cp,
        scratch_types=[
            [pltpu.VMEM((chunk,), jnp.float32)] * 2,
            [pltpu.VMEM((chunk,), jnp.float32)] * 2,
            [pltpu.VMEM((chunk,), jnp.float32)] * 2,
            [pltpu.VMEM((n_gw, _GW), jnp.int32)] * 2,
            [pltpu.VMEM((idx_len,), jnp.int32)] * 2,
            [pltpu.VMEM((n_rows, 8), jnp.float32)] * 2,
            [pltpu.VMEM((idx_len * 2,), jnp.float32)] * 2,
            [pltpu.SemaphoreType.DMA] * 2,
            [pltpu.SemaphoreType.DMA] * 2,
        ],
    )
    def run(px_hbm, py_hbm, pz_hbm, tab_hbm, out_hbm,
            px_v, py_v, pz_v, idx_v, col_v, rows_v, out_v, gsem, osem):
        wid = (lax.axis_index("subcore").astype(jnp.int32) * np.int32(_NC)
               + lax.axis_index("core").astype(jnp.int32))
        lane = lax.iota(jnp.int32, 16)
        lane2 = lane * np.int32(2)
        lane16 = lane * np.int32(16)
        lane32 = lane * np.int32(32)

        def stage_a(ci, b):
            base = wid * np.int32(pw) + ci * np.int32(chunk)
            pltpu.sync_copy(px_hbm.at[pl.ds(base, chunk)], px_v[b])
            pltpu.sync_copy(py_hbm.at[pl.ds(base, chunk)], py_v[b])
            pltpu.sync_copy(pz_hbm.at[pl.ds(base, chunk)], pz_v[b])

            @functools.partial(_loop32, chunk // _LANES)
            def _grp(g):
                goff = g * np.int32(_LANES)
                x = px_v[b][pl.ds(goff, _LANES)]
                y = py_v[b][pl.ds(goff, _LANES)]
                z = pz_v[b][pl.ds(goff, _LANES)]
                one = jnp.float32(1.0)
                half = jnp.float32(0.5)
                zero = jnp.float32(0.0)
                pnx = jnp.minimum(jnp.maximum((x + one) * half, zero), one)
                pny = jnp.minimum(jnp.maximum((y + one) * half, zero), one)
                pnz = jnp.minimum(jnp.maximum((z + one) * half, zero), one)
                gb16 = g * np.int32(_LANES * _NUM_LEVELS)
                gb32 = g * np.int32(_LANES * _NUM_LEVELS * 2)
                for lvl in range(_NUM_LEVELS):
                    gx = (pnx * res_m1[lvl]).astype(jnp.int32)
                    gy = (pny * res_m1[lvl]).astype(jnp.int32)
                    gz = (pnz * res_m1[lvl]).astype(jnp.int32)
                    h = gx ^ (gy * _P1) ^ (gz * _P2)
                    h = h & jnp.int32(_MASK)
                    r0 = (lax.shift_right_logical(h, jnp.int32(3))
                          + lax.shift_left(
                              lax.shift_right_logical(h, jnp.int32(7)),
                              jnp.int32(4))
                          + jnp.int32(lvl * _LROWS))
                    r1 = r0 + jnp.int32(16)
                    off = h & jnp.int32(7)
                    pos2 = lane32 + (gb32 + np.int32(2 * lvl))
                    row = lax.shift_right_logical(pos2, jnp.int32(7))
                    col = pos2 & jnp.int32(_GW - 1)
                    plsc.store_scatter(idx_v[b], [row, col], r0)
                    plsc.store_scatter(idx_v[b], [row, col + jnp.int32(1)], r1)
                    pos = lane16 + (gb16 + np.int32(lvl))
                    plsc.store_scatter(col_v[b], [pos], off)

            @functools.partial(_loop32, n_gw)
            def _fire(ji):
                pltpu.async_copy(
                    tab_hbm.at[idx_v[b].at[ji]],
                    rows_v[b].at[pl.ds(ji * np.int32(_GW), _GW)], gsem[b])

        def stage_b(ci, b):
            base = wid * np.int32(pw) + ci * np.int32(chunk)

            @functools.partial(_loop32, n_gw)
            def _drain(ji):
                pltpu.make_async_copy(
                    tab_hbm.at[idx_v[b].at[ji]],
                    rows_v[b].at[pl.ds(ji * np.int32(_GW), _GW)],
                    gsem[b]).wait()

            @pl.when(ci >= jnp.int32(2))
            def _():
                pltpu.make_async_copy(
                    out_v[b],
                    out_hbm.at[pl.ds(jnp.int32(0), idx_len * 2)],
                    osem[b]).wait()

            posf0 = (lax.shift_left(
                         lax.shift_right_logical(lane2, jnp.int32(3)),
                         jnp.int32(10))
                     + lax.shift_left(lane2 & jnp.int32(7), jnp.int32(7)))

            @functools.partial(_loop32, idx_len // _LANES)
            def _compact(j):
                j16 = j * np.int32(_LANES)
                rowj = lane2 + j * np.int32(2 * _LANES)
                colj = col_v[b][pl.ds(j16, _LANES)]
                f0 = plsc.load_gather(rows_v[b], [rowj, colj])
                f1 = plsc.load_gather(rows_v[b], [rowj + jnp.int32(1), colj])
                p0 = posf0 + j
                plsc.store_scatter(out_v[b], [p0], f0)
                plsc.store_scatter(out_v[b], [p0 + jnp.int32(128)], f1)

            tc = lax.shift_right_logical(base, jnp.int32(7))
            n_blocks = n_points // 128
            for tr in range(4):
                pltpu.async_copy(
                    out_v[b].at[pl.ds(np.int32(tr * 1024), 1024)],
                    out_hbm.at[pl.ds(tc * np.int32(1024)
                                     + np.int32(tr * n_blocks * 1024), 1024)],
                    osem[b])

        stage_a(jnp.int32(0), 0)

        @functools.partial(_loop32, n_chunks // 2)
        def _pair(i):
            ci0 = i * np.int32(2)
            ci1 = ci0 + jnp.int32(1)
            stage_a(ci1, 1)
            stage_b(ci0, 0)

            @pl.when(ci1 + jnp.int32(1) < jnp.int32(n_chunks))
            def _():
                stage_a(ci1 + jnp.int32(1), 0)

            stage_b(ci1, 1)

        for b in range(2):
            pltpu.make_async_copy(
                out_v[b], out_hbm.at[pl.ds(jnp.int32(0), idx_len * 2)],
                osem[b]).wait()

    return run(px, py, pz, table_w8)


def kernel(points, tables):
    n = points.shape[0]
    px = points[:, 0]
    py = points[:, 1]
    pz = points[:, 2]
    table_w8 = (tables.reshape(_NUM_LEVELS, _TABLE_SIZE // _GW, _GW, 2)
                .transpose(0, 1, 3, 2)
                .reshape(_NUM_LEVELS * _TABLE_SIZE * 2 // 8, 8))
    out = _hash_encode_sc(px, py, pz, table_w8, n, chunk=128)
    return (out.reshape(4, n // 128, 8, 128)
            .transpose(1, 3, 0, 2)
            .reshape(n, _NUM_LEVELS * 2))


# speedup vs baseline: 82.7438x; 1.1697x over previous
import dataclasses
import functools

import jax
import jax.numpy as jnp
import numpy as np
from jax import lax
from jax.experimental import pallas as pl
from jax.experimental.pallas import tpu as pltpu
from jax.experimental.pallas import tpu_sc as plsc

_NUM_LEVELS = 16
_LOG2_HASHMAP = 19
_TABLE_SIZE = 1 << _LOG2_HASHMAP
_MASK = _TABLE_SIZE - 1
_BASE_RES = 16
_FINEST_RES = 2048
_P1 = np.uint32(2654435761).astype(np.int32)
_P2 = np.int32(805459861)

_NC = 2
_NS = 16
_NW = _NC * _NS
_LANES = 16
_GW = 128
_LROWS = _TABLE_SIZE * 2 // 8


def _loop32(n, body, unroll=1):
    if unroll == 1:
        lax.fori_loop(jnp.int32(0), jnp.int32(n), lambda i, _: body(i), None)
        return
    assert n % unroll == 0

    def _body(i, _):
        ib = i * np.int32(unroll)
        for k in range(unroll):
            body(ib + jnp.int32(k))

    lax.fori_loop(jnp.int32(0), jnp.int32(n // unroll), _body, None)


def _resolutions():
    res = []
    for i in range(_NUM_LEVELS):
        r = int(np.floor(_BASE_RES * np.exp(
            i * np.log(_FINEST_RES / _BASE_RES) / (_NUM_LEVELS - 1))))
        res.append(r)
    return res


def _hash_encode_sc(px, py, pz, table_w8, n_points, chunk):
    pw = n_points // _NW
    n_chunks = pw // chunk
    idx_len = chunk * _NUM_LEVELS
    n_rows = idx_len * 2
    n_gw = n_rows // _GW
    res_m1 = [np.float32(r - 1) for r in _resolutions()]

    mesh = plsc.VectorSubcoreMesh(core_axis_name="core",
                                  subcore_axis_name="subcore",
                                  num_cores=_NC, num_subcores=_NS)
    cp = pltpu.CompilerParams()
    if "needs_layout_passes" in pltpu.CompilerParams.__dataclass_fields__:
        cp = dataclasses.replace(cp, needs_layout_passes=False)
    if "use_tc_tiling_on_sc" in pltpu.CompilerParams.__dataclass_fields__:
        cp = dataclasses.replace(cp, use_tc_tiling_on_sc=False)

    @functools.partial(
        pl.kernel,
        out_type=jax.ShapeDtypeStruct((n_points * _NUM_LEVELS * 2,),
                                      jnp.float32),
        mesh=mesh,
        compiler_params=cp,
        scratch_types=[
            [pltpu.VMEM((chunk,), jnp.float32)] * 2,
            [pltpu.VMEM((chunk,), jnp.float32)] * 2,
            [pltpu.VMEM((chunk,), jnp.float32)] * 2,
            [pltpu.VMEM((n_gw, _GW), jnp.int32)] * 2,
            [pltpu.VMEM((idx_len,), jnp.int32)] * 2,
            [pltpu.VMEM((n_rows, 8), jnp.float32)] * 2,
            [pltpu.VMEM((idx_len * 2,), jnp.float32)] * 2,
            [pltpu.SemaphoreType.DMA] * 2,
            [pltpu.SemaphoreType.DMA] * 2,
        ],
    )
    def run(px_hbm, py_hbm, pz_hbm, tab_hbm, out_hbm,
            px_v, py_v, pz_v, idx_v, col_v, rows_v, out_v, gsem, osem):
        wid = (lax.axis_index("subcore").astype(jnp.int32) * np.int32(_NC)
               + lax.axis_index("core").astype(jnp.int32))
        lane = lax.iota(jnp.int32, 16)
        lane2 = lane * np.int32(2)
        lane16 = lane * np.int32(16)
        lane32 = lane * np.int32(32)

        def stage_a(ci, b):
            base = wid * np.int32(pw) + ci * np.int32(chunk)
            pltpu.sync_copy(px_hbm.at[pl.ds(base, chunk)], px_v[b])
            pltpu.sync_copy(py_hbm.at[pl.ds(base, chunk)], py_v[b])
            pltpu.sync_copy(pz_hbm.at[pl.ds(base, chunk)], pz_v[b])

            @functools.partial(_loop32, chunk // _LANES)
            def _grp(g):
                goff = g * np.int32(_LANES)
                x = px_v[b][pl.ds(goff, _LANES)]
                y = py_v[b][pl.ds(goff, _LANES)]
                z = pz_v[b][pl.ds(goff, _LANES)]
                one = jnp.float32(1.0)
                half = jnp.float32(0.5)
                zero = jnp.float32(0.0)
                pnx = jnp.minimum(jnp.maximum((x + one) * half, zero), one)
                pny = jnp.minimum(jnp.maximum((y + one) * half, zero), one)
                pnz = jnp.minimum(jnp.maximum((z + one) * half, zero), one)
                g4 = g * np.int32(4)
                g256 = g * np.int32(256)
                for lvl in range(_NUM_LEVELS):
                    gx = (pnx * res_m1[lvl]).astype(jnp.int32)
                    gy = (pny * res_m1[lvl]).astype(jnp.int32)
                    gz = (pnz * res_m1[lvl]).astype(jnp.int32)
                    h = gx ^ (gy * _P1) ^ (gz * _P2)
                    h = h & jnp.int32(_MASK)
                    r0 = (lax.shift_right_logical(h, jnp.int32(3))
                          + lax.shift_left(
                              lax.shift_right_logical(h, jnp.int32(7)),
                              jnp.int32(4))
                          + jnp.int32(lvl * _LROWS))
                    r1 = r0 + jnp.int32(16)
                    off = h & jnp.int32(7)
                    irow = g4 + np.int32(lvl >> 2)
                    icol = (lvl & 3) * 32
                    idx_v[b][irow, pl.ds(icol, _LANES)] = r0
                    idx_v[b][irow, pl.ds(icol + _LANES, _LANES)] = r1
                    col_v[b][pl.ds(g256 + np.int32(lvl * _LANES), _LANES)] = off

            @functools.partial(_loop32, n_gw)
            def _fire(ji):
                pltpu.async_copy(
                    tab_hbm.at[idx_v[b].at[ji]],
                    rows_v[b].at[pl.ds(ji * np.int32(_GW), _GW)], gsem[b])

        def stage_b(ci, b):
            base = wid * np.int32(pw) + ci * np.int32(chunk)

            @functools.partial(_loop32, n_gw)
            def _drain(ji):
                pltpu.make_async_copy(
                    tab_hbm.at[idx_v[b].at[ji]],
                    rows_v[b].at[pl.ds(ji * np.int32(_GW), _GW)],
                    gsem[b]).wait()

            @pl.when(ci >= jnp.int32(2))
            def _():
                pltpu.make_async_copy(
                    out_v[b],
                    out_hbm.at[pl.ds(jnp.int32(0), idx_len * 2)],
                    osem[b]).wait()

            posf0 = (lax.shift_left(
                         lax.shift_right_logical(lane2, jnp.int32(3)),
                         jnp.int32(10))
                     + lax.shift_left(lane2 & jnp.int32(7), jnp.int32(7)))

            @functools.partial(_loop32, idx_len // _LANES, unroll=4)
            def _compact(j):
                jlo = j & jnp.int32(15)
                jg = lax.shift_right_logical(j, jnp.int32(4))
                colj = plsc.load_gather(
                    col_v[b], [lane16 + (jg * np.int32(256) + jlo)])
                rowj = lane32 + (jg * np.int32(512) + jlo)
                f0 = plsc.load_gather(rows_v[b], [rowj, colj])
                f1 = plsc.load_gather(rows_v[b],
                                      [rowj + jnp.int32(16), colj])
                p0 = posf0 + j
                plsc.store_scatter(out_v[b], [p0], f0)
                plsc.store_scatter(out_v[b], [p0 + jnp.int32(128)], f1)

            tc = lax.shift_right_logical(base, jnp.int32(7))
            n_blocks = n_points // 128
            for tr in range(4):
                pltpu.async_copy(
                    out_v[b].at[pl.ds(np.int32(tr * 1024), 1024)],
                    out_hbm.at[pl.ds(tc * np.int32(1024)
                                     + np.int32(tr * n_blocks * 1024), 1024)],
                    osem[b])

        stage_a(jnp.int32(0), 0)

        @functools.partial(_loop32, n_chunks // 2)
        def _pair(i):
            ci0 = i * np.int32(2)
            ci1 = ci0 + jnp.int32(1)
            stage_a(ci1, 1)
            stage_b(ci0, 0)

            @pl.when(ci1 + jnp.int32(1) < jnp.int32(n_chunks))
            def _():
                stage_a(ci1 + jnp.int32(1), 0)

            stage_b(ci1, 1)

        for b in range(2):
            pltpu.make_async_copy(
                out_v[b], out_hbm.at[pl.ds(jnp.int32(0), idx_len * 2)],
                osem[b]).wait()

    return run(px, py, pz, table_w8)


def kernel(points, tables):
    n = points.shape[0]
    px = points[:, 0]
    py = points[:, 1]
    pz = points[:, 2]
    table_w8 = (tables.reshape(_NUM_LEVELS, _TABLE_SIZE // _GW, _GW, 2)
                .transpose(0, 1, 3, 2)
                .reshape(_NUM_LEVELS * _TABLE_SIZE * 2 // 8, 8))
    out = _hash_encode_sc(px, py, pz, table_w8, n, chunk=128)
    return (out.reshape(4, n // 128, 8, 128)
            .transpose(1, 3, 0, 2)
            .reshape(n, _NUM_LEVELS * 2))


# speedup vs baseline: 94.8786x; 1.1467x over previous
import dataclasses
import functools

import jax
import jax.numpy as jnp
import numpy as np
from jax import lax
from jax.experimental import pallas as pl
from jax.experimental.pallas import tpu as pltpu
from jax.experimental.pallas import tpu_sc as plsc

_NUM_LEVELS = 16
_LOG2_HASHMAP = 19
_TABLE_SIZE = 1 << _LOG2_HASHMAP
_MASK = _TABLE_SIZE - 1
_BASE_RES = 16
_FINEST_RES = 2048
_P1 = np.uint32(2654435761).astype(np.int32)
_P2 = np.int32(805459861)

_NC = 2
_NS = 16
_NW = _NC * _NS
_LANES = 16
_GW = 128
_LROWS = _TABLE_SIZE * 2 // 8


def _loop32(n, body, unroll=1):
    if unroll == 1:
        lax.fori_loop(jnp.int32(0), jnp.int32(n), lambda i, _: body(i), None)
        return
    assert n % unroll == 0

    def _body(i, _):
        ib = i * np.int32(unroll)
        for k in range(unroll):
            body(ib + jnp.int32(k))

    lax.fori_loop(jnp.int32(0), jnp.int32(n // unroll), _body, None)


def _resolutions():
    res = []
    for i in range(_NUM_LEVELS):
        r = int(np.floor(_BASE_RES * np.exp(
            i * np.log(_FINEST_RES / _BASE_RES) / (_NUM_LEVELS - 1))))
        res.append(r)
    return res


def _hash_encode_sc(px, py, pz, table_w8, n_points, chunk):
    pw = n_points // _NW
    n_chunks = pw // chunk
    idx_len = chunk * _NUM_LEVELS
    n_rows = idx_len * 2
    n_gw = n_rows // _GW
    res_m1 = [np.float32(r - 1) for r in _resolutions()]

    mesh = plsc.VectorSubcoreMesh(core_axis_name="core",
                                  subcore_axis_name="subcore",
                                  num_cores=_NC, num_subcores=_NS)
    cp = pltpu.CompilerParams()
    if "needs_layout_passes" in pltpu.CompilerParams.__dataclass_fields__:
        cp = dataclasses.replace(cp, needs_layout_passes=False)
    if "use_tc_tiling_on_sc" in pltpu.CompilerParams.__dataclass_fields__:
        cp = dataclasses.replace(cp, use_tc_tiling_on_sc=False)

    @functools.partial(
        pl.kernel,
        out_type=jax.ShapeDtypeStruct((n_points * _NUM_LEVELS * 2,),
                                      jnp.float32),
        mesh=mesh,
        compiler_params=cp,
        scratch_types=[
            [pltpu.VMEM((chunk,), jnp.float32)] * 2,
            [pltpu.VMEM((chunk,), jnp.float32)] * 2,
            [pltpu.VMEM((chunk,), jnp.float32)] * 2,
            [pltpu.VMEM((n_gw, _GW), jnp.int32)] * 2,
            [pltpu.VMEM((idx_len,), jnp.int32)] * 2,
            [pltpu.VMEM((n_rows, 8), jnp.float32)] * 2,
            [pltpu.VMEM((idx_len * 2,), jnp.float32)] * 2,
            [pltpu.SemaphoreType.DMA] * 2,
            [pltpu.SemaphoreType.DMA] * 2,
            [pltpu.SemaphoreType.DMA] * 2,
        ],
    )
    def run(px_hbm, py_hbm, pz_hbm, tab_hbm, out_hbm,
            px_v, py_v, pz_v, idx_v, col_v, rows_v, out_v, gsem, osem, psem):
        wid = (lax.axis_index("subcore").astype(jnp.int32) * np.int32(_NC)
               + lax.axis_index("core").astype(jnp.int32))
        lane = lax.iota(jnp.int32, 16)
        lane2 = lane * np.int32(2)
        lane16 = lane * np.int32(16)
        lane32 = lane * np.int32(32)

        def fire_pts(ci, b):
            base = wid * np.int32(pw) + ci * np.int32(chunk)
            pltpu.async_copy(px_hbm.at[pl.ds(base, chunk)], px_v[b], psem[b])
            pltpu.async_copy(py_hbm.at[pl.ds(base, chunk)], py_v[b], psem[b])
            pltpu.async_copy(pz_hbm.at[pl.ds(base, chunk)], pz_v[b], psem[b])

        def wait_pts(ci, b):
            base = wid * np.int32(pw) + ci * np.int32(chunk)
            for v in (px_v, py_v, pz_v):
                pltpu.make_async_copy(
                    px_hbm.at[pl.ds(base, chunk)], v[b], psem[b]).wait()

        def stage_a(ci, b):
            base = wid * np.int32(pw) + ci * np.int32(chunk)
            wait_pts(ci, b)

            @functools.partial(_loop32, chunk // _LANES, unroll=2)
            def _grp(g):
                goff = g * np.int32(_LANES)
                x = px_v[b][pl.ds(goff, _LANES)]
                y = py_v[b][pl.ds(goff, _LANES)]
                z = pz_v[b][pl.ds(goff, _LANES)]
                one = jnp.float32(1.0)
                half = jnp.float32(0.5)
                zero = jnp.float32(0.0)
                pnx = jnp.minimum(jnp.maximum((x + one) * half, zero), one)
                pny = jnp.minimum(jnp.maximum((y + one) * half, zero), one)
                pnz = jnp.minimum(jnp.maximum((z + one) * half, zero), one)
                g4 = g * np.int32(4)
                g256 = g * np.int32(256)
                for lvl in range(_NUM_LEVELS):
                    gx = (pnx * res_m1[lvl]).astype(jnp.int32)
                    gy = (pny * res_m1[lvl]).astype(jnp.int32)
                    gz = (pnz * res_m1[lvl]).astype(jnp.int32)
                    h = gx ^ (gy * _P1) ^ (gz * _P2)
                    h = h & jnp.int32(_MASK)
                    r0 = (lax.shift_right_logical(h, jnp.int32(3))
                          + lax.shift_left(
                              lax.shift_right_logical(h, jnp.int32(7)),
                              jnp.int32(4))
                          + jnp.int32(lvl * _LROWS))
                    r1 = r0 + jnp.int32(16)
                    off = h & jnp.int32(7)
                    irow = g4 + np.int32(lvl >> 2)
                    icol = (lvl & 3) * 32
                    idx_v[b][irow, pl.ds(icol, _LANES)] = r0
                    idx_v[b][irow, pl.ds(icol + _LANES, _LANES)] = r1
                    col_v[b][pl.ds(g256 + np.int32(lvl * _LANES), _LANES)] = off

            @functools.partial(_loop32, n_gw, unroll=4)
            def _fire(ji):
                pltpu.async_copy(
                    tab_hbm.at[idx_v[b].at[ji]],
                    rows_v[b].at[pl.ds(ji * np.int32(_GW), _GW)], gsem[b])

            @pl.when(ci + jnp.int32(2) < jnp.int32(n_chunks))
            def _():
                fire_pts(ci + jnp.int32(2), b)

        def stage_b(ci, b):
            base = wid * np.int32(pw) + ci * np.int32(chunk)

            @functools.partial(_loop32, n_gw, unroll=4)
            def _drain(ji):
                pltpu.make_async_copy(
                    tab_hbm.at[idx_v[b].at[ji]],
                    rows_v[b].at[pl.ds(ji * np.int32(_GW), _GW)],
                    gsem[b]).wait()

            @pl.when(ci >= jnp.int32(2))
            def _():
                pltpu.make_async_copy(
                    out_v[b],
                    out_hbm.at[pl.ds(jnp.int32(0), idx_len * 2)],
                    osem[b]).wait()

            posf0 = (lax.shift_left(
                         lax.shift_right_logical(lane2, jnp.int32(3)),
                         jnp.int32(10))
                     + lax.shift_left(lane2 & jnp.int32(7), jnp.int32(7)))

            @functools.partial(_loop32, idx_len // _LANES, unroll=4)
            def _compact(j):
                jlo = j & jnp.int32(15)
                jg = lax.shift_right_logical(j, jnp.int32(4))
                colj = plsc.load_gather(
                    col_v[b], [lane16 + (jg * np.int32(256) + jlo)])
                rowj = lane32 + (jg * np.int32(512) + jlo)
                f0 = plsc.load_gather(rows_v[b], [rowj, colj])
                f1 = plsc.load_gather(rows_v[b],
                                      [rowj + jnp.int32(16), colj])
                p0 = posf0 + j
                plsc.store_scatter(out_v[b], [p0], f0)
                plsc.store_scatter(out_v[b], [p0 + jnp.int32(128)], f1)

            tc = lax.shift_right_logical(base, jnp.int32(7))
            n_blocks = n_points // 128
            for tr in range(4):
                pltpu.async_copy(
                    out_v[b].at[pl.ds(np.int32(tr * 1024), 1024)],
                    out_hbm.at[pl.ds(tc * np.int32(1024)
                                     + np.int32(tr * n_blocks * 1024), 1024)],
                    osem[b])

        fire_pts(jnp.int32(0), 0)
        fire_pts(jnp.int32(1), 1)
        stage_a(jnp.int32(0), 0)

        @functools.partial(_loop32, n_chunks // 2)
        def _pair(i):
            ci0 = i * np.int32(2)
            ci1 = ci0 + jnp.int32(1)
            stage_a(ci1, 1)
            stage_b(ci0, 0)

            @pl.when(ci1 + jnp.int32(1) < jnp.int32(n_chunks))
            def _():
                stage_a(ci1 + jnp.int32(1), 0)

            stage_b(ci1, 1)

        for b in range(2):
            pltpu.make_async_copy(
                out_v[b], out_hbm.at[pl.ds(jnp.int32(0), idx_len * 2)],
                osem[b]).wait()

    return run(px, py, pz, table_w8)


def kernel(points, tables):
    n = points.shape[0]
    px = points[:, 0]
    py = points[:, 1]
    pz = points[:, 2]
    table_w8 = (tables.reshape(_NUM_LEVELS, _TABLE_SIZE // _GW, _GW, 2)
                .transpose(0, 1, 3, 2)
                .reshape(_NUM_LEVELS * _TABLE_SIZE * 2 // 8, 8))
    out = _hash_encode_sc(px, py, pz, table_w8, n, chunk=128)
    return (out.reshape(4, n // 128, 8, 128)
            .transpose(1, 3, 0, 2)
            .reshape(n, _NUM_LEVELS * 2))
